# Initial kernel scaffold; baseline (speedup 1.0000x reference)
import functools

import jax
import jax.numpy as jnp
from jax import lax
from jax.experimental import pallas as pl
from jax.experimental.pallas import tpu as pltpu
from jax.experimental.pallas import tpu_sc as plsc

B, S = 1024, 20
N = B * S
N2V_DIM = 128
T_FEAT, ED = 4, 8
T2V_DIM = 32
PROJ_IN = 216
TARGET = 64

_INFO = plsc.get_sparse_core_info()
NC, NS = _INFO.num_cores, _INFO.num_subcores
NW = NC * NS
TOK_PER_W = N // NW
CHUNK = 128
NCHUNK = TOK_PER_W // CHUNK


def _sc_gather_body(n2v_hbm, bt_hbm, ev_hbm, eq_hbm,
                    idx_n2v_hbm, idx_bt_hbm, idx_ev_hbm, idx_eq_hbm,
                    out_spat, out_bt, out_ev, out_eq,
                    idx_v, big_v, small_v, sem):
    wid = lax.axis_index("s") * NC + lax.axis_index("c")
    base = wid * TOK_PER_W

    def one_table(table_hbm, idx_hbm, out_hbm, rows_v):
        pltpu.sync_copy(idx_hbm.at[pl.ds(wid * NCHUNK, NCHUNK)], idx_v)
        for j in range(NCHUNK):
            pltpu.async_copy(table_hbm.at[idx_v.at[j]], rows_v, sem).wait()
            pltpu.sync_copy(rows_v, out_hbm.at[pl.ds(base + j * CHUNK, CHUNK)])

    one_table(n2v_hbm, idx_n2v_hbm, out_spat, big_v)
    one_table(bt_hbm, idx_bt_hbm, out_bt, small_v)
    one_table(ev_hbm, idx_ev_hbm, out_ev, small_v)
    one_table(eq_hbm, idx_eq_hbm, out_eq, small_v)


def _sc_gather(n2v, btab, etab, qtab, idx_n2v, idx_bt, idx_ev, idx_eq):
    mesh = plsc.VectorSubcoreMesh(core_axis_name="c", subcore_axis_name="s")
    fn = pl.kernel(
        _sc_gather_body,
        out_type=[
            jax.ShapeDtypeStruct((N, N2V_DIM), jnp.float32),
            jax.ShapeDtypeStruct((N, 16), jnp.float32),
            jax.ShapeDtypeStruct((N, 16), jnp.float32),
            jax.ShapeDtypeStruct((N, 16), jnp.float32),
        ],
        mesh=mesh,
        scratch_types=[
            pltpu.VMEM((NCHUNK, CHUNK), jnp.int32),
            pltpu.VMEM((CHUNK, N2V_DIM), jnp.float32),
            pltpu.VMEM((CHUNK, 16), jnp.float32),
            pltpu.SemaphoreType.DMA,
        ],
    )
    return fn(n2v, btab, etab, qtab, idx_n2v, idx_bt, idx_ev, idx_eq)


BN = 2048


def _tc_body(spat_ref, bt_ref, ev_ref, eq_ref, tf_ref, popc_ref, counts0_ref,
             wf_ref, bf_ref, popw_ref, popb_ref, projw_ref, projb_ref,
             out_ref):
    i = pl.program_id(0)
    f32 = jnp.float32
    hi = lax.Precision.HIGHEST

    r4 = lax.broadcasted_iota(jnp.int32, (T_FEAT, T2V_DIM), 0)
    c4 = lax.broadcasted_iota(jnp.int32, (T_FEAT, T2V_DIM), 1)
    expand = (c4 // ED == r4).astype(f32)
    tfr = jnp.dot(tf_ref[...], expand, preferred_element_type=f32,
                  precision=hi)
    aff = tfr * wf_ref[...] + bf_ref[...]
    lane = lax.broadcasted_iota(jnp.int32, (BN, T2V_DIM), 1)
    temporal = jnp.where(lane % ED == 0, aff, jnp.sin(aff))

    csum = jnp.sum(counts0_ref[...], axis=0, keepdims=True)
    rr = lax.broadcasted_iota(jnp.int32, (BN, S), 0) + i * BN
    cc = lax.broadcasted_iota(jnp.int32, (BN, S), 1)
    onehot = ((rr % S) == cc).astype(f32)
    scale = lax.dot_general(onehot, csum, (((1,), (1,)), ((), ())),
                            precision=hi)
    building = bt_ref[...] * scale

    pope = popc_ref[...] * popw_ref[...] + popb_ref[...]

    combined = jnp.concatenate(
        [spat_ref[...], temporal, building, pope, ev_ref[...], eq_ref[...]],
        axis=1)
    out_ref[...] = jnp.dot(combined, projw_ref[...],
                           preferred_element_type=f32,
                           precision=hi) + projb_ref[...]


def _tc_fused(spat, btrow, evrow, eqrow, tf, popc, counts0,
              wf, bf, popw, popb, projw, projb):
    grid = (N // BN,)
    row_spec = lambda w: pl.BlockSpec((BN, w), lambda i: (i, 0))
    full = lambda a: pl.BlockSpec(a.shape, lambda i: (0,) * a.ndim)
    return pl.pallas_call(
        _tc_body,
        grid=grid,
        in_specs=[
            row_spec(N2V_DIM), row_spec(16), row_spec(16), row_spec(16),
            row_spec(T_FEAT), row_spec(1), full(counts0),
            full(wf), full(bf), full(popw), full(popb),
            full(projw), full(projb),
        ],
        out_specs=pl.BlockSpec((BN, TARGET), lambda i: (i, 0)),
        out_shape=jax.ShapeDtypeStruct((N, TARGET), jnp.float32),
    )(spat, btrow, evrow, eqrow, tf, popc, counts0,
      wf, bf, popw, popb, projw, projb)


def kernel(neighborhood_ids, time_features, building_type_ids,
           building_counts, population, event_type_ids, equipment_ids,
           node2vec_table, t2v_weight, t2v_bias, building_table,
           pop_W, pop_b, event_table, equip_table, proj_W, proj_b):
    nrow = N // CHUNK
    idx_n2v = neighborhood_ids.reshape(nrow, CHUNK)
    idx_bt = building_type_ids[:, :, 0].reshape(nrow, CHUNK)
    idx_ev = event_type_ids.reshape(nrow, CHUNK)
    idx_eq = equipment_ids.reshape(nrow, CHUNK)

    spat, btrow, evrow, eqrow = _sc_gather(
        node2vec_table, building_table, event_table, equip_table,
        idx_n2v, idx_bt, idx_ev, idx_eq)

    out = _tc_fused(
        spat, btrow, evrow, eqrow,
        time_features.reshape(N, T_FEAT),
        population.reshape(N, 1),
        building_counts[:, :, 0],
        t2v_weight.reshape(1, T2V_DIM),
        t2v_bias.reshape(1, T2V_DIM),
        pop_W, pop_b.reshape(1, 8),
        proj_W, proj_b.reshape(1, TARGET))
    return out.reshape(B, S, TARGET)


# speedup vs baseline: 5.5363x; 5.5363x over previous
import jax
import jax.numpy as jnp
from jax import lax
from jax.experimental import pallas as pl
from jax.experimental.pallas import tpu as pltpu
from jax.experimental.pallas import tpu_sc as plsc

B, S = 1024, 20
N = B * S
N2V_DIM = 128
T_FEAT, ED = 4, 8
T2V_DIM = 32
SMALL_D = 16
SMALL_V = 1000
TARGET = 64

NC, NS = 2, 16
NW = NC * NS
TOK_PER_W = N // NW
CHUNK = 128
NCHUNK = TOK_PER_W // CHUNK
NGRP = TOK_PER_W // 16


def _sc_gather_body(n2v_hbm, bt_hbm, ev_hbm, eq_hbm,
                    idx_n2v_hbm, idx_bt_hbm, idx_ev_hbm, idx_eq_hbm,
                    out_spat, out_bt, out_ev, out_eq,
                    idx_v, big_v, tab_v, outt_v, sem):
    wid = lax.axis_index("s") * NC + lax.axis_index("c")
    base = wid * TOK_PER_W

    pltpu.sync_copy(idx_n2v_hbm.at[pl.ds(base, TOK_PER_W)], idx_v)
    for j in range(NCHUNK):
        pltpu.async_copy(
            n2v_hbm.at[idx_v.at[pl.ds(j * CHUNK, CHUNK)]], big_v, sem).wait()
        pltpu.sync_copy(big_v, out_spat.at[pl.ds(base + j * CHUNK, CHUNK)])

    def small_table(tab_hbm, idx_hbm, out_hbm):
        pltpu.sync_copy(tab_hbm, tab_v)
        pltpu.sync_copy(idx_hbm.at[pl.ds(base, TOK_PER_W)], idx_v)

        def grp(g, _):
            off = pl.multiple_of(g * 16, 16)
            ids = idx_v[pl.ds(off, 16)] * SMALL_D
            for k in range(SMALL_D):
                outt_v[k, pl.ds(off, 16)] = plsc.load_gather(tab_v, [ids + k])
            return _

        lax.fori_loop(0, NGRP, grp, None)
        pltpu.sync_copy(outt_v, out_hbm.at[:, pl.ds(base, TOK_PER_W)])

    small_table(bt_hbm, idx_bt_hbm, out_bt)
    small_table(ev_hbm, idx_ev_hbm, out_ev)
    small_table(eq_hbm, idx_eq_hbm, out_eq)


def _sc_gather(n2v, btab, etab, qtab, idx_n2v, idx_bt, idx_ev, idx_eq):
    mesh = plsc.VectorSubcoreMesh(core_axis_name="c", subcore_axis_name="s")
    fn = pl.kernel(
        _sc_gather_body,
        out_type=[
            jax.ShapeDtypeStruct((N, N2V_DIM), jnp.float32),
            jax.ShapeDtypeStruct((SMALL_D, N), jnp.float32),
            jax.ShapeDtypeStruct((SMALL_D, N), jnp.float32),
            jax.ShapeDtypeStruct((SMALL_D, N), jnp.float32),
        ],
        mesh=mesh,
        compiler_params=pltpu.CompilerParams(needs_layout_passes=False),
        scratch_types=[
            pltpu.VMEM((TOK_PER_W,), jnp.int32),
            pltpu.VMEM((CHUNK, N2V_DIM), jnp.float32),
            pltpu.VMEM((SMALL_V * SMALL_D,), jnp.float32),
            pltpu.VMEM((SMALL_D, TOK_PER_W), jnp.float32),
            pltpu.SemaphoreType.DMA,
        ],
    )
    return fn(n2v, btab, etab, qtab, idx_n2v, idx_bt, idx_ev, idx_eq)


BN = 2048


def _tc_body(spat_ref, btt_ref, evt_ref, eqt_ref, tf_ref, popc_ref,
             counts0_ref, wf_ref, bf_ref, popw_ref, popb_ref,
             w1_ref, w2_ref, w3_ref, w4_ref, w5_ref, w6_ref, projb_ref,
             out_ref):
    i = pl.program_id(0)
    f32 = jnp.float32
    hi = lax.Precision.HIGHEST
    c0 = (((0,), (0,)), ((), ()))

    r4 = lax.broadcasted_iota(jnp.int32, (T_FEAT, T2V_DIM), 0)
    c4 = lax.broadcasted_iota(jnp.int32, (T_FEAT, T2V_DIM), 1)
    expand = (c4 // ED == r4).astype(f32)
    tfr = jnp.dot(tf_ref[...], expand, preferred_element_type=f32,
                  precision=hi)
    aff = tfr * wf_ref[...] + bf_ref[...]
    lane = lax.broadcasted_iota(jnp.int32, (BN, T2V_DIM), 1)
    temporal = jnp.where(lane % ED == 0, aff, jnp.sin(aff))

    csum = jnp.sum(counts0_ref[...], axis=0, keepdims=True)
    rr = lax.broadcasted_iota(jnp.int32, (BN, S), 0) + i * BN
    cc = lax.broadcasted_iota(jnp.int32, (BN, S), 1)
    onehot = ((rr % S) == cc).astype(f32)
    scale = lax.dot_general(onehot, csum, (((1,), (1,)), ((), ())),
                            precision=hi)

    pw2 = jnp.dot(popw_ref[...], w4_ref[...], preferred_element_type=f32,
                  precision=hi)
    pb2 = jnp.dot(popb_ref[...], w4_ref[...], preferred_element_type=f32,
                  precision=hi)

    acc = jnp.dot(spat_ref[...], w1_ref[...], preferred_element_type=f32,
                  precision=hi)
    acc += jnp.dot(temporal, w2_ref[...], preferred_element_type=f32,
                   precision=hi)
    acc += lax.dot_general(btt_ref[...], w3_ref[...], c0,
                           precision=hi) * scale
    acc += popc_ref[...] * pw2 + pb2
    acc += lax.dot_general(evt_ref[...], w5_ref[...], c0, precision=hi)
    acc += lax.dot_general(eqt_ref[...], w6_ref[...], c0, precision=hi)
    out_ref[...] = acc + projb_ref[...]


def _tc_fused(spat, btt, evt, eqt, tf, popc, counts0,
              wf, bf, popw, popb, w1, w2, w3, w4, w5, w6, projb):
    grid = (N // BN,)
    row_spec = lambda w: pl.BlockSpec((BN, w), lambda i: (i, 0))
    colt_spec = pl.BlockSpec((SMALL_D, BN), lambda i: (0, i))
    full = lambda a: pl.BlockSpec(a.shape, lambda i: (0,) * a.ndim)
    return pl.pallas_call(
        _tc_body,
        grid=grid,
        in_specs=[
            row_spec(N2V_DIM), colt_spec, colt_spec, colt_spec,
            row_spec(T_FEAT), row_spec(1), full(counts0),
            full(wf), full(bf), full(popw), full(popb),
            full(w1), full(w2), full(w3), full(w4), full(w5), full(w6),
            full(projb),
        ],
        out_specs=pl.BlockSpec((BN, TARGET), lambda i: (i, 0)),
        out_shape=jax.ShapeDtypeStruct((N, TARGET), jnp.float32),
    )(spat, btt, evt, eqt, tf, popc, counts0,
      wf, bf, popw, popb, w1, w2, w3, w4, w5, w6, projb)


def kernel(neighborhood_ids, time_features, building_type_ids,
           building_counts, population, event_type_ids, equipment_ids,
           node2vec_table, t2v_weight, t2v_bias, building_table,
           pop_W, pop_b, event_table, equip_table, proj_W, proj_b):
    idx_n2v = neighborhood_ids.reshape(N)
    idx_bt = building_type_ids[:, :, 0].reshape(N)
    idx_ev = event_type_ids.reshape(N)
    idx_eq = equipment_ids.reshape(N)

    spat, btt, evt, eqt = _sc_gather(
        node2vec_table,
        building_table.reshape(SMALL_V * SMALL_D),
        event_table.reshape(SMALL_V * SMALL_D),
        equip_table.reshape(SMALL_V * SMALL_D),
        idx_n2v, idx_bt, idx_ev, idx_eq)

    out = _tc_fused(
        spat, btt, evt, eqt,
        time_features.reshape(N, T_FEAT),
        population.reshape(N, 1),
        building_counts[:, :, 0],
        t2v_weight.reshape(1, T2V_DIM),
        t2v_bias.reshape(1, T2V_DIM),
        pop_W, pop_b.reshape(1, 8),
        proj_W[0:128], proj_W[128:160], proj_W[160:176],
        proj_W[176:184], proj_W[184:200], proj_W[200:216],
        proj_b.reshape(1, TARGET))
    return out.reshape(B, S, TARGET)


# speedup vs baseline: 8.1081x; 1.4645x over previous
import jax
import jax.numpy as jnp
from jax import lax
from jax.experimental import pallas as pl
from jax.experimental.pallas import tpu as pltpu
from jax.experimental.pallas import tpu_sc as plsc

B, S = 1024, 20
N = B * S
N2V_DIM = 128
T_FEAT, ED = 4, 8
T2V_DIM = 32
SMALL_D = 16
SMALL_V = 1000
TARGET = 64

NC, NS = 2, 16
NW = NC * NS
TOK_PER_W = N // NW
CHUNK = 128
NCHUNK = TOK_PER_W // CHUNK
NGRP = TOK_PER_W // 16


def _sc_gather_body(n2v_hbm, bt_hbm, ev_hbm, eq_hbm,
                    idx_n2v_hbm, idx_bt_hbm, idx_ev_hbm, idx_eq_hbm,
                    out_spat, out_bt, out_ev, out_eq,
                    idx_v, big_v, tab_v, outt_v, sem):
    wid = lax.axis_index("s") * NC + lax.axis_index("c")
    base = wid * TOK_PER_W

    pltpu.sync_copy(idx_n2v_hbm.at[pl.ds(base, TOK_PER_W)], idx_v)
    for j in range(NCHUNK):
        pltpu.async_copy(
            n2v_hbm.at[idx_v.at[pl.ds(j * CHUNK, CHUNK)]], big_v, sem).wait()
        pltpu.sync_copy(big_v, out_spat.at[pl.ds(base + j * CHUNK, CHUNK)])

    def small_table(tab_hbm, idx_hbm, out_hbm):
        pltpu.sync_copy(tab_hbm, tab_v)
        pltpu.sync_copy(idx_hbm.at[pl.ds(base, TOK_PER_W)], idx_v)

        def grp(g, _):
            off = pl.multiple_of(g * 16, 16)
            ids = idx_v[pl.ds(off, 16)] * SMALL_D
            for k in range(SMALL_D):
                outt_v[k, pl.ds(off, 16)] = plsc.load_gather(tab_v, [ids + k])
            return _

        lax.fori_loop(0, NGRP, grp, None)
        pltpu.sync_copy(outt_v, out_hbm.at[:, pl.ds(base, TOK_PER_W)])

    small_table(bt_hbm, idx_bt_hbm, out_bt)
    small_table(ev_hbm, idx_ev_hbm, out_ev)
    small_table(eq_hbm, idx_eq_hbm, out_eq)


def _sc_gather(n2v, btab, etab, qtab, idx_n2v, idx_bt, idx_ev, idx_eq):
    mesh = plsc.VectorSubcoreMesh(core_axis_name="c", subcore_axis_name="s")
    fn = pl.kernel(
        _sc_gather_body,
        out_type=[
            jax.ShapeDtypeStruct((N, N2V_DIM), jnp.float32),
            jax.ShapeDtypeStruct((SMALL_D, N), jnp.float32),
            jax.ShapeDtypeStruct((SMALL_D, N), jnp.float32),
            jax.ShapeDtypeStruct((SMALL_D, N), jnp.float32),
        ],
        mesh=mesh,
        compiler_params=pltpu.CompilerParams(needs_layout_passes=False),
        scratch_types=[
            pltpu.VMEM((TOK_PER_W,), jnp.int32),
            pltpu.VMEM((CHUNK, N2V_DIM), jnp.float32),
            pltpu.VMEM((SMALL_V * SMALL_D,), jnp.float32),
            pltpu.VMEM((SMALL_D, TOK_PER_W), jnp.float32),
            pltpu.SemaphoreType.DMA,
        ],
    )
    return fn(n2v, btab, etab, qtab, idx_n2v, idx_bt, idx_ev, idx_eq)


BN = 2048

_PI_HI = 3.14159274101257324
_PI_LO = -8.74227765734758577e-08


def _fast_sin(x):
    n = jnp.round(x * (1.0 / 3.14159265358979))
    r = x - n * _PI_HI
    r = r - n * _PI_LO
    r = jnp.clip(r, -1.6, 1.6)
    s = r * r
    p = -2.50507586e-08
    p = p * s + 2.75573143e-06
    p = p * s + -1.98412701e-04
    p = p * s + 8.33333377e-03
    p = p * s + -1.66666672e-01
    p = r + r * (s * p)
    odd = jnp.round(n * 0.5) * 2.0 != n
    return jnp.where(odd, -p, p)


def _tc_body(spat_ref, btt_ref, evt_ref, eqt_ref, tf_ref, popc_ref,
             counts0_ref, wf_ref, bf_ref, popw_ref, popb_ref,
             w1_ref, w2_ref, w3_ref, w4_ref, w5_ref, w6_ref, projb_ref,
             out_ref):
    i = pl.program_id(0)
    f32 = jnp.float32
    hi = lax.Precision.HIGHEST
    c0 = (((0,), (0,)), ((), ()))

    r4 = lax.broadcasted_iota(jnp.int32, (T_FEAT, T2V_DIM), 0)
    c4 = lax.broadcasted_iota(jnp.int32, (T_FEAT, T2V_DIM), 1)
    expand = (c4 // ED == r4).astype(f32)
    tfr = jnp.dot(tf_ref[...], expand, preferred_element_type=f32,
                  precision=hi)
    aff = tfr * wf_ref[...] + bf_ref[...]
    lane = lax.broadcasted_iota(jnp.int32, (BN, T2V_DIM), 1)
    temporal = jnp.where(lane % ED == 0, aff, _fast_sin(aff))

    csum = jnp.sum(counts0_ref[...], axis=0, keepdims=True)
    rr = lax.broadcasted_iota(jnp.int32, (BN, S), 0) + i * BN
    cc = lax.broadcasted_iota(jnp.int32, (BN, S), 1)
    onehot = ((rr % S) == cc).astype(f32)
    scale = lax.dot_general(onehot, csum, (((1,), (1,)), ((), ())),
                            precision=hi)

    pw2 = jnp.dot(popw_ref[...], w4_ref[...], preferred_element_type=f32,
                  precision=hi)
    pb2 = jnp.dot(popb_ref[...], w4_ref[...], preferred_element_type=f32,
                  precision=hi)

    acc = jnp.dot(spat_ref[...], w1_ref[...], preferred_element_type=f32)
    acc += jnp.dot(temporal, w2_ref[...], preferred_element_type=f32)
    acc += lax.dot_general(btt_ref[...], w3_ref[...], c0,
                           preferred_element_type=f32) * scale
    acc += popc_ref[...] * pw2 + pb2
    acc += lax.dot_general(evt_ref[...], w5_ref[...], c0,
                           preferred_element_type=f32)
    acc += lax.dot_general(eqt_ref[...], w6_ref[...], c0,
                           preferred_element_type=f32)
    out_ref[...] = acc + projb_ref[...]


def _tc_fused(spat, btt, evt, eqt, tf, popc, counts0,
              wf, bf, popw, popb, w1, w2, w3, w4, w5, w6, projb):
    grid = (N // BN,)
    row_spec = lambda w: pl.BlockSpec((BN, w), lambda i: (i, 0))
    colt_spec = pl.BlockSpec((SMALL_D, BN), lambda i: (0, i))
    full = lambda a: pl.BlockSpec(a.shape, lambda i: (0,) * a.ndim)
    return pl.pallas_call(
        _tc_body,
        grid=grid,
        in_specs=[
            row_spec(N2V_DIM), colt_spec, colt_spec, colt_spec,
            row_spec(T_FEAT), row_spec(1), full(counts0),
            full(wf), full(bf), full(popw), full(popb),
            full(w1), full(w2), full(w3), full(w4), full(w5), full(w6),
            full(projb),
        ],
        out_specs=pl.BlockSpec((BN, TARGET), lambda i: (i, 0)),
        out_shape=jax.ShapeDtypeStruct((N, TARGET), jnp.float32),
        compiler_params=pltpu.CompilerParams(
            fuse_transposed_lhs_in_matmul=True),
    )(spat, btt, evt, eqt, tf, popc, counts0,
      wf, bf, popw, popb, w1, w2, w3, w4, w5, w6, projb)


def kernel(neighborhood_ids, time_features, building_type_ids,
           building_counts, population, event_type_ids, equipment_ids,
           node2vec_table, t2v_weight, t2v_bias, building_table,
           pop_W, pop_b, event_table, equip_table, proj_W, proj_b):
    idx_n2v = neighborhood_ids.reshape(N)
    idx_bt = building_type_ids[:, :, 0].reshape(N)
    idx_ev = event_type_ids.reshape(N)
    idx_eq = equipment_ids.reshape(N)

    spat, btt, evt, eqt = _sc_gather(
        node2vec_table,
        building_table.reshape(SMALL_V * SMALL_D),
        event_table.reshape(SMALL_V * SMALL_D),
        equip_table.reshape(SMALL_V * SMALL_D),
        idx_n2v, idx_bt, idx_ev, idx_eq)

    out = _tc_fused(
        spat, btt, evt, eqt,
        time_features.reshape(N, T_FEAT),
        population.reshape(N, 1),
        building_counts[:, :, 0],
        t2v_weight.reshape(1, T2V_DIM),
        t2v_bias.reshape(1, T2V_DIM),
        pop_W, pop_b.reshape(1, 8),
        proj_W[0:128], proj_W[128:160], proj_W[160:176],
        proj_W[176:184], proj_W[184:200], proj_W[200:216],
        proj_b.reshape(1, TARGET))
    return out.reshape(B, S, TARGET)


# speedup vs baseline: 11.1058x; 1.3697x over previous
import functools

import jax
import jax.numpy as jnp
from jax import lax
from jax.experimental import pallas as pl
from jax.experimental.pallas import tpu as pltpu
from jax.experimental.pallas import tpu_sc as plsc

B, S = 1024, 20
N = B * S
NH = N // 2
SH = S // 2
N2V_DIM = 128
T_FEAT, ED = 4, 8
T2V_DIM = 32
SMALL_D = 16
SMALL_V = 1000
TARGET = 64

NC, NS = 2, 16
NW = NC * NS
TOKW_B = NH // NW
BIG_OFFS = (0, 128, 256)
BIG_LENS = (128, 128, 64)
NWS = 16
TOKW_S = NH // NWS
NGRP = TOKW_S // 16


def _sc_gather_body(n2v_hbm, bt_hbm, ev_hbm, eq_hbm,
                    idx_n2v_hbm, idx_bt_hbm, idx_ev_hbm, idx_eq_hbm,
                    out_spat, out_bt, out_ev, out_eq,
                    idx_v, idxs_v, big_v, tab_v, outt_v, sem):
    wid = lax.axis_index("s") * NC + lax.axis_index("c")
    bbase = wid * TOKW_B

    pltpu.sync_copy(idx_n2v_hbm.at[pl.ds(bbase, TOKW_B)], idx_v)
    big_copies = [
        pltpu.async_copy(
            n2v_hbm.at[idx_v.at[pl.ds(o, l)]], big_v.at[pl.ds(o, l)], sem)
        for o, l in zip(BIG_OFFS, BIG_LENS)
    ]

    @pl.when(wid < NWS)
    def _small_tables():
        sbase = wid * TOKW_S

        def small_table(tab_hbm, idx_hbm, out_hbm):
            pltpu.sync_copy(tab_hbm, tab_v)
            pltpu.sync_copy(idx_hbm.at[pl.ds(sbase, TOKW_S)], idxs_v)

            def grp(g, _):
                off = pl.multiple_of(g * 16, 16)
                ids = idxs_v[pl.ds(off, 16)] * SMALL_D
                for k in range(SMALL_D):
                    outt_v[k, pl.ds(off, 16)] = plsc.load_gather(
                        tab_v, [ids + k])
                return _

            lax.fori_loop(0, NGRP, grp, None)
            pltpu.sync_copy(outt_v, out_hbm.at[:, pl.ds(sbase, TOKW_S)])

        small_table(bt_hbm, idx_bt_hbm, out_bt)
        small_table(ev_hbm, idx_ev_hbm, out_ev)
        small_table(eq_hbm, idx_eq_hbm, out_eq)

    for (o, l), c in zip(zip(BIG_OFFS, BIG_LENS), big_copies):
        c.wait()
        pltpu.sync_copy(big_v.at[pl.ds(o, l)],
                        out_spat.at[pl.ds(bbase + o, l)])


def _sc_gather(n2v, btab, etab, qtab, idx_n2v, idx_bt, idx_ev, idx_eq):
    mesh = plsc.VectorSubcoreMesh(core_axis_name="c", subcore_axis_name="s")
    fn = pl.kernel(
        _sc_gather_body,
        out_type=[
            jax.ShapeDtypeStruct((NH, N2V_DIM), jnp.float32),
            jax.ShapeDtypeStruct((SMALL_D, NH), jnp.float32),
            jax.ShapeDtypeStruct((SMALL_D, NH), jnp.float32),
            jax.ShapeDtypeStruct((SMALL_D, NH), jnp.float32),
        ],
        mesh=mesh,
        compiler_params=pltpu.CompilerParams(needs_layout_passes=False),
        scratch_types=[
            pltpu.VMEM((TOKW_B,), jnp.int32),
            pltpu.VMEM((TOKW_S,), jnp.int32),
            pltpu.VMEM((TOKW_B, N2V_DIM), jnp.float32),
            pltpu.VMEM((SMALL_V * SMALL_D,), jnp.float32),
            pltpu.VMEM((SMALL_D, TOKW_S), jnp.float32),
            pltpu.SemaphoreType.DMA,
        ],
    )
    return fn(n2v, btab, etab, qtab, idx_n2v, idx_bt, idx_ev, idx_eq)


_PI_HI = 3.14159274101257324
_PI_LO = -8.74227765734758577e-08


def _fast_sin(x):
    n = jnp.round(x * (1.0 / 3.14159265358979))
    r = x - n * _PI_HI
    r = r - n * _PI_LO
    r = jnp.clip(r, -1.6, 1.6)
    s = r * r
    p = -2.50507586e-08
    p = p * s + 2.75573143e-06
    p = p * s + -1.98412701e-04
    p = p * s + 8.33333377e-03
    p = p * s + -1.66666672e-01
    p = r + r * (s * p)
    odd = jnp.round(n * 0.5) * 2.0 != n
    return jnp.where(odd, -p, p)


def _tc_body(spat_ref, btt_ref, evt_ref, eqt_ref, tf_ref, popc_ref,
             counts_ref, wf_ref, bf_ref, popw_ref, popb_ref,
             w1t_ref, w2t_ref, w3t_ref, w4t_ref, w5t_ref, w6t_ref,
             projb_ref, out_ref):
    f32 = jnp.float32

    tf_blk = tf_ref[...]
    tfr = jnp.concatenate(
        [jnp.broadcast_to(tf_blk[t:t + 1, :], (ED, B)) for t in range(T_FEAT)],
        axis=0)
    aff = tfr * wf_ref[...] + bf_ref[...]
    row = lax.broadcasted_iota(jnp.int32, (T2V_DIM, B), 0)
    temporal = jnp.where(row % ED == 0, aff, _fast_sin(aff))

    csum_s = jnp.sum(counts_ref[...])

    pw2 = lax.dot_general(w4t_ref[...], popw_ref[...],
                          (((1,), (0,)), ((), ())),
                          preferred_element_type=f32)
    pb2 = lax.dot_general(w4t_ref[...], popb_ref[...],
                          (((1,), (0,)), ((), ())),
                          preferred_element_type=f32)

    cR = (((1,), (0,)), ((), ()))
    cT = (((1,), (1,)), ((), ()))
    acc = lax.dot_general(w1t_ref[...], spat_ref[...], cT,
                          preferred_element_type=f32)
    acc += lax.dot_general(w2t_ref[...], temporal, cR,
                           preferred_element_type=f32)
    acc += lax.dot_general(w3t_ref[...], btt_ref[...], cR,
                           preferred_element_type=f32) * csum_s
    acc += pw2 * popc_ref[...] + pb2
    acc += lax.dot_general(w5t_ref[...], evt_ref[...], cR,
                           preferred_element_type=f32)
    acc += lax.dot_general(w6t_ref[...], eqt_ref[...], cR,
                           preferred_element_type=f32)
    out_ref[0] = acc + projb_ref[...]


def _tc_body_aliased(prev_ref, *refs):
    del prev_ref
    _tc_body(*refs)


def _tc_fused(half, prev, spat, btt, evt, eqt, tfT, popcT, countsT,
              wf, bf, popw, popb, w1t, w2t, w3t, w4t, w5t, w6t, projb):
    grid = (SH,)
    off = half * SH
    col_spec = lambda h: pl.BlockSpec((h, B), lambda i: (0, i))
    dense_spec = lambda h: pl.BlockSpec((h, B), lambda i: (0, i + off))
    full = lambda a: pl.BlockSpec(a.shape, lambda i: (0,) * a.ndim)
    in_specs = [
        pl.BlockSpec((B, N2V_DIM), lambda i: (i, 0)),
        col_spec(SMALL_D), col_spec(SMALL_D), col_spec(SMALL_D),
        dense_spec(T_FEAT), dense_spec(1),
        pl.BlockSpec((1, 1, B), lambda i: (i + off, 0, 0)),
        full(wf), full(bf), full(popw), full(popb),
        full(w1t), full(w2t), full(w3t), full(w4t), full(w5t),
        full(w6t), full(projb),
    ]
    args = (spat, btt, evt, eqt, tfT, popcT, countsT,
            wf, bf, popw, popb, w1t, w2t, w3t, w4t, w5t, w6t, projb)
    kwargs = {}
    body = _tc_body
    if prev is not None:
        in_specs = [pl.BlockSpec(memory_space=pltpu.MemorySpace.HBM)] \
            + in_specs
        args = (prev,) + args
        kwargs["input_output_aliases"] = {0: 0}
        body = _tc_body_aliased
    return pl.pallas_call(
        body,
        grid=grid,
        in_specs=in_specs,
        out_specs=pl.BlockSpec((1, TARGET, B), lambda i: (i + off, 0, 0)),
        out_shape=jax.ShapeDtypeStruct((S, TARGET, B), jnp.float32),
        compiler_params=pltpu.CompilerParams(
            fuse_transposed_lhs_in_matmul=True),
        **kwargs,
    )(*args)


def kernel(neighborhood_ids, time_features, building_type_ids,
           building_counts, population, event_type_ids, equipment_ids,
           node2vec_table, t2v_weight, t2v_bias, building_table,
           pop_W, pop_b, event_table, equip_table, proj_W, proj_b):
    idx_n2v = neighborhood_ids.reshape(B, S).T.reshape(N)
    idx_bt = building_type_ids[:, :, 0].T.reshape(N)
    idx_ev = event_type_ids.reshape(B, S).T.reshape(N)
    idx_eq = equipment_ids.reshape(B, S).T.reshape(N)

    btab = building_table.reshape(SMALL_V * SMALL_D)
    etab = event_table.reshape(SMALL_V * SMALL_D)
    qtab = equip_table.reshape(SMALL_V * SMALL_D)

    halves = [
        _sc_gather(node2vec_table, btab, etab, qtab,
                   idx_n2v[h * NH:(h + 1) * NH],
                   idx_bt[h * NH:(h + 1) * NH],
                   idx_ev[h * NH:(h + 1) * NH],
                   idx_eq[h * NH:(h + 1) * NH])
        for h in range(2)
    ]

    wt = proj_W.T
    tfT = jnp.transpose(time_features, (2, 1, 0)).reshape(T_FEAT, N)
    popcT = population[:, :, 0].T.reshape(1, N)
    countsT = building_counts[:, :, 0].T.reshape(S, 1, B)
    dense_args = (tfT, popcT, countsT,
                  t2v_weight.reshape(T2V_DIM, 1),
                  t2v_bias.reshape(T2V_DIM, 1),
                  pop_W.reshape(8, 1), pop_b.reshape(8, 1),
                  wt[:, 0:128], wt[:, 128:160], wt[:, 160:176],
                  wt[:, 176:184], wt[:, 184:200], wt[:, 200:216],
                  proj_b.reshape(TARGET, 1))

    out = None
    for h in range(2):
        spat, btt, evt, eqt = halves[h]
        out = _tc_fused(h, out, spat, btt, evt, eqt, *dense_args)
    return jnp.transpose(out, (2, 0, 1))


# speedup vs baseline: 14.5239x; 1.3078x over previous
import jax
import jax.numpy as jnp
from jax import lax
from jax.experimental import pallas as pl
from jax.experimental.pallas import tpu as pltpu
from jax.experimental.pallas import tpu_sc as plsc

B, S = 1024, 20
N = B * S
N2V_DIM = 128
T_FEAT, ED = 4, 8
T2V_DIM = 32
SMALL_D = 16
SMALL_V = 1000
TARGET = 64

NC, NS = 2, 16
NW = NC * NS
TOK_PER_W = N // NW
CHUNK = 128
NCHUNK = TOK_PER_W // CHUNK
NGRP = TOK_PER_W // 16


def _sc_gather_body(n2v_hbm, bt_hbm, ev_hbm, eq_hbm,
                    idx_n2v_hbm, idx_bt_hbm, idx_ev_hbm, idx_eq_hbm,
                    out_spat, out_bt, out_ev, out_eq,
                    idx_v, idxs_v, big_v, tab_v, outt_v, sem):
    wid = lax.axis_index("s") * NC + lax.axis_index("c")
    base = wid * TOK_PER_W

    pltpu.sync_copy(idx_n2v_hbm.at[pl.ds(base, TOK_PER_W)], idx_v)
    big_copies = [
        pltpu.async_copy(
            n2v_hbm.at[idx_v.at[pl.ds(j * CHUNK, CHUNK)]],
            big_v.at[pl.ds(j * CHUNK, CHUNK)], sem)
        for j in range(NCHUNK)
    ]

    def small_table(tab_hbm, idx_hbm, out_hbm):
        pltpu.sync_copy(tab_hbm, tab_v)
        pltpu.sync_copy(idx_hbm.at[pl.ds(base, TOK_PER_W)], idxs_v)

        def grp(g, _):
            off = pl.multiple_of(g * 16, 16)
            ids = idxs_v[pl.ds(off, 16)] * SMALL_D
            for k in range(SMALL_D):
                outt_v[k, pl.ds(off, 16)] = plsc.load_gather(tab_v, [ids + k])
            return _

        lax.fori_loop(0, NGRP, grp, None)
        pltpu.sync_copy(outt_v, out_hbm.at[:, pl.ds(base, TOK_PER_W)])

    small_table(bt_hbm, idx_bt_hbm, out_bt)
    small_table(ev_hbm, idx_ev_hbm, out_ev)
    small_table(eq_hbm, idx_eq_hbm, out_eq)

    for j, c in enumerate(big_copies):
        c.wait()
        pltpu.sync_copy(big_v.at[pl.ds(j * CHUNK, CHUNK)],
                        out_spat.at[pl.ds(base + j * CHUNK, CHUNK)])


def _sc_gather(n2v, btab, etab, qtab, idx_n2v, idx_bt, idx_ev, idx_eq):
    mesh = plsc.VectorSubcoreMesh(core_axis_name="c", subcore_axis_name="s")
    fn = pl.kernel(
        _sc_gather_body,
        out_type=[
            jax.ShapeDtypeStruct((N, N2V_DIM), jnp.float32),
            jax.ShapeDtypeStruct((SMALL_D, N), jnp.float32),
            jax.ShapeDtypeStruct((SMALL_D, N), jnp.float32),
            jax.ShapeDtypeStruct((SMALL_D, N), jnp.float32),
        ],
        mesh=mesh,
        compiler_params=pltpu.CompilerParams(needs_layout_passes=False),
        scratch_types=[
            pltpu.VMEM((TOK_PER_W,), jnp.int32),
            pltpu.VMEM((TOK_PER_W,), jnp.int32),
            pltpu.VMEM((TOK_PER_W, N2V_DIM), jnp.float32),
            pltpu.VMEM((SMALL_V * SMALL_D,), jnp.float32),
            pltpu.VMEM((SMALL_D, TOK_PER_W), jnp.float32),
            pltpu.SemaphoreType.DMA,
        ],
    )
    return fn(n2v, btab, etab, qtab, idx_n2v, idx_bt, idx_ev, idx_eq)


_PI_HI = 3.14159274101257324
_PI_LO = -8.74227765734758577e-08


def _fast_sin(x):
    n = jnp.round(x * (1.0 / 3.14159265358979))
    r = x - n * _PI_HI
    r = r - n * _PI_LO
    r = jnp.clip(r, -1.6, 1.6)
    s = r * r
    p = -2.50507586e-08
    p = p * s + 2.75573143e-06
    p = p * s + -1.98412701e-04
    p = p * s + 8.33333377e-03
    p = p * s + -1.66666672e-01
    p = r + r * (s * p)
    odd = jnp.round(n * 0.5) * 2.0 != n
    return jnp.where(odd, -p, p)


SB = 2


def _tc_body(spat_ref, btt_ref, evt_ref, eqt_ref, tf_ref, popc_ref,
             counts_ref, wf_ref, bf_ref, popw_ref, popb_ref,
             w1t_ref, w2t_ref, w3t_ref, w4t_ref, w5t_ref, w6t_ref,
             projb_ref, out_ref):
    f32 = jnp.float32

    pw2 = lax.dot_general(w4t_ref[...], popw_ref[...],
                          (((1,), (0,)), ((), ())),
                          preferred_element_type=f32)
    pb2 = lax.dot_general(w4t_ref[...], popb_ref[...],
                          (((1,), (0,)), ((), ())),
                          preferred_element_type=f32)

    cR = (((1,), (0,)), ((), ()))
    cT = (((1,), (1,)), ((), ()))

    for u in range(SB):
        c = pl.ds(u * B, B)

        tf_blk = tf_ref[:, c]
        tfr = jnp.concatenate(
            [jnp.broadcast_to(tf_blk[t:t + 1, :], (ED, B))
             for t in range(T_FEAT)], axis=0)
        aff = tfr * wf_ref[...] + bf_ref[...]
        row = lax.broadcasted_iota(jnp.int32, (T2V_DIM, B), 0)
        temporal = jnp.where(row % ED == 0, aff, _fast_sin(aff))

        csum_s = jnp.sum(counts_ref[u])

        acc = lax.dot_general(w1t_ref[...], spat_ref[pl.ds(u * B, B), :],
                              cT, preferred_element_type=f32)
        acc += lax.dot_general(w2t_ref[...], temporal, cR,
                               preferred_element_type=f32)
        acc += lax.dot_general(w3t_ref[...], btt_ref[:, c], cR,
                               preferred_element_type=f32) * csum_s
        acc += pw2 * popc_ref[:, c] + pb2
        acc += lax.dot_general(w5t_ref[...], evt_ref[:, c], cR,
                               preferred_element_type=f32)
        acc += lax.dot_general(w6t_ref[...], eqt_ref[:, c], cR,
                               preferred_element_type=f32)
        out_ref[u] = acc + projb_ref[...]


def _tc_fused(spat, btt, evt, eqt, tfT, popcT, countsT,
              wf, bf, popw, popb, w1t, w2t, w3t, w4t, w5t, w6t, projb):
    grid = (S // SB,)
    col_spec = lambda h: pl.BlockSpec((h, SB * B), lambda i: (0, i))
    full = lambda a: pl.BlockSpec(a.shape, lambda i: (0,) * a.ndim)
    return pl.pallas_call(
        _tc_body,
        grid=grid,
        in_specs=[
            pl.BlockSpec((SB * B, N2V_DIM), lambda i: (i, 0)),
            col_spec(SMALL_D), col_spec(SMALL_D), col_spec(SMALL_D),
            col_spec(T_FEAT), col_spec(1),
            pl.BlockSpec((SB, 1, B), lambda i: (i, 0, 0)),
            full(wf), full(bf), full(popw), full(popb),
            full(w1t), full(w2t), full(w3t), full(w4t), full(w5t),
            full(w6t), full(projb),
        ],
        out_specs=pl.BlockSpec((SB, TARGET, B), lambda i: (i, 0, 0)),
        out_shape=jax.ShapeDtypeStruct((S, TARGET, B), jnp.float32),
        compiler_params=pltpu.CompilerParams(
            fuse_transposed_lhs_in_matmul=True),
    )(spat, btt, evt, eqt, tfT, popcT, countsT,
      wf, bf, popw, popb, w1t, w2t, w3t, w4t, w5t, w6t, projb)


def kernel(neighborhood_ids, time_features, building_type_ids,
           building_counts, population, event_type_ids, equipment_ids,
           node2vec_table, t2v_weight, t2v_bias, building_table,
           pop_W, pop_b, event_table, equip_table, proj_W, proj_b):
    idx_n2v = neighborhood_ids.reshape(B, S).T.reshape(N)
    idx_bt = building_type_ids[:, :, 0].T.reshape(N)
    idx_ev = event_type_ids.reshape(B, S).T.reshape(N)
    idx_eq = equipment_ids.reshape(B, S).T.reshape(N)

    spat, btt, evt, eqt = _sc_gather(
        node2vec_table,
        building_table.reshape(SMALL_V * SMALL_D),
        event_table.reshape(SMALL_V * SMALL_D),
        equip_table.reshape(SMALL_V * SMALL_D),
        idx_n2v, idx_bt, idx_ev, idx_eq)

    wt = proj_W.T
    out = _tc_fused(
        spat, btt, evt, eqt,
        jnp.transpose(time_features, (2, 1, 0)).reshape(T_FEAT, N),
        population[:, :, 0].T.reshape(1, N),
        building_counts[:, :, 0].T.reshape(S, 1, B),
        t2v_weight.reshape(T2V_DIM, 1),
        t2v_bias.reshape(T2V_DIM, 1),
        pop_W.reshape(8, 1), pop_b.reshape(8, 1),
        wt[:, 0:128], wt[:, 128:160], wt[:, 160:176],
        wt[:, 176:184], wt[:, 184:200], wt[:, 200:216],
        proj_b.reshape(TARGET, 1))
    return jnp.transpose(out, (2, 0, 1))


# speedup vs baseline: 14.9122x; 1.0267x over previous
import jax
import jax.numpy as jnp
from jax import lax
from jax.experimental import pallas as pl
from jax.experimental.pallas import tpu as pltpu
from jax.experimental.pallas import tpu_sc as plsc

B, S = 1024, 20
N = B * S
N2V_DIM = 128
T_FEAT, ED = 4, 8
T2V_DIM = 32
SMALL_D = 16
SMALL_V = 1000
TARGET = 64

NC, NS = 2, 16
NW = NC * NS
TOK_PER_W = N // NW
CHUNK = 128
NCHUNK = TOK_PER_W // CHUNK
NGRP = TOK_PER_W // 16


def _sc_gather_body(n2v_hbm, bt_hbm, ev_hbm, eq_hbm,
                    idx_n2v_hbm, idx_bt_hbm, idx_ev_hbm, idx_eq_hbm,
                    out_spat, out_bt, out_ev, out_eq,
                    idx_v, idxs_v, big_v, tab_v, outt_v, sem):
    wid = lax.axis_index("s") * NC + lax.axis_index("c")
    base = wid * TOK_PER_W

    pltpu.sync_copy(idx_n2v_hbm.at[pl.ds(base, TOK_PER_W)], idx_v)
    big_copies = [
        pltpu.async_copy(
            n2v_hbm.at[idx_v.at[pl.ds(j * CHUNK, CHUNK)]],
            big_v.at[pl.ds(j * CHUNK, CHUNK)], sem)
        for j in range(NCHUNK)
    ]

    def small_table(tab_hbm, idx_hbm, out_hbm):
        pltpu.sync_copy(tab_hbm, tab_v)
        pltpu.sync_copy(idx_hbm.at[pl.ds(base, TOK_PER_W)], idxs_v)

        def grp(g, _):
            off = pl.multiple_of(g * 16, 16)
            ids = idxs_v[pl.ds(off, 16)] * SMALL_D
            for k in range(SMALL_D):
                outt_v[k, pl.ds(off, 16)] = plsc.load_gather(tab_v, [ids + k])
            return _

        lax.fori_loop(0, NGRP, grp, None)
        pltpu.sync_copy(outt_v, out_hbm.at[:, pl.ds(base, TOK_PER_W)])

    small_table(bt_hbm, idx_bt_hbm, out_bt)
    small_table(ev_hbm, idx_ev_hbm, out_ev)
    small_table(eq_hbm, idx_eq_hbm, out_eq)

    for j, c in enumerate(big_copies):
        c.wait()
        pltpu.sync_copy(big_v.at[pl.ds(j * CHUNK, CHUNK)],
                        out_spat.at[pl.ds(base + j * CHUNK, CHUNK)])


def _sc_gather(n2v, btab, etab, qtab, idx_n2v, idx_bt, idx_ev, idx_eq):
    mesh = plsc.VectorSubcoreMesh(core_axis_name="c", subcore_axis_name="s")
    fn = pl.kernel(
        _sc_gather_body,
        out_type=[
            jax.ShapeDtypeStruct((N, N2V_DIM), jnp.float32),
            jax.ShapeDtypeStruct((SMALL_D, N), jnp.float32),
            jax.ShapeDtypeStruct((SMALL_D, N), jnp.float32),
            jax.ShapeDtypeStruct((SMALL_D, N), jnp.float32),
        ],
        mesh=mesh,
        compiler_params=pltpu.CompilerParams(needs_layout_passes=False),
        scratch_types=[
            pltpu.VMEM((TOK_PER_W,), jnp.int32),
            pltpu.VMEM((TOK_PER_W,), jnp.int32),
            pltpu.VMEM((TOK_PER_W, N2V_DIM), jnp.float32),
            pltpu.VMEM((SMALL_V * SMALL_D,), jnp.float32),
            pltpu.VMEM((SMALL_D, TOK_PER_W), jnp.float32),
            pltpu.SemaphoreType.DMA,
        ],
    )
    return fn(n2v, btab, etab, qtab, idx_n2v, idx_bt, idx_ev, idx_eq)


_PI_HI = 3.14159274101257324
_PI_LO = -8.74227765734758577e-08


def _fast_sin(x):
    n = jnp.round(x * (1.0 / 3.14159265358979))
    r = x - n * _PI_HI
    r = r - n * _PI_LO
    r = jnp.clip(r, -1.6, 1.6)
    s = r * r
    p = -2.50507586e-08
    p = p * s + 2.75573143e-06
    p = p * s + -1.98412701e-04
    p = p * s + 8.33333377e-03
    p = p * s + -1.66666672e-01
    p = r + r * (s * p)
    odd = jnp.round(n * 0.5) * 2.0 != n
    return jnp.where(odd, -p, p)


SB = 4


def _tc_body(spat_ref, btt_ref, evt_ref, eqt_ref, tf_ref, popc_ref,
             counts_ref, wf_ref, bf_ref, popw_ref, popb_ref,
             w1t_ref, w2t_ref, w3t_ref, w4t_ref, w5t_ref, w6t_ref,
             projb_ref, out_ref):
    f32 = jnp.float32

    pw2 = lax.dot_general(w4t_ref[...], popw_ref[...],
                          (((1,), (0,)), ((), ())),
                          preferred_element_type=f32)
    pb2 = lax.dot_general(w4t_ref[...], popb_ref[...],
                          (((1,), (0,)), ((), ())),
                          preferred_element_type=f32)

    cR = (((1,), (0,)), ((), ()))
    cT = (((1,), (1,)), ((), ()))

    for u in range(SB):
        c = pl.ds(u * B, B)

        tf_blk = tf_ref[:, c]
        tfr = jnp.concatenate(
            [jnp.broadcast_to(tf_blk[t:t + 1, :], (ED, B))
             for t in range(T_FEAT)], axis=0)
        aff = tfr * wf_ref[...] + bf_ref[...]
        row = lax.broadcasted_iota(jnp.int32, (T2V_DIM, B), 0)
        temporal = jnp.where(row % ED == 0, aff, _fast_sin(aff))

        csum_s = jnp.sum(counts_ref[u])

        acc = lax.dot_general(w1t_ref[...], spat_ref[pl.ds(u * B, B), :],
                              cT, preferred_element_type=f32)
        acc += lax.dot_general(w2t_ref[...], temporal, cR,
                               preferred_element_type=f32)
        acc += lax.dot_general(w3t_ref[...], btt_ref[:, c], cR,
                               preferred_element_type=f32) * csum_s
        acc += pw2 * popc_ref[:, c] + pb2
        acc += lax.dot_general(w5t_ref[...], evt_ref[:, c], cR,
                               preferred_element_type=f32)
        acc += lax.dot_general(w6t_ref[...], eqt_ref[:, c], cR,
                               preferred_element_type=f32)
        out_ref[u] = acc + projb_ref[...]


def _tc_fused(spat, btt, evt, eqt, tfT, popcT, countsT,
              wf, bf, popw, popb, w1t, w2t, w3t, w4t, w5t, w6t, projb):
    grid = (S // SB,)
    col_spec = lambda h: pl.BlockSpec((h, SB * B), lambda i: (0, i))
    full = lambda a: pl.BlockSpec(a.shape, lambda i: (0,) * a.ndim)
    return pl.pallas_call(
        _tc_body,
        grid=grid,
        in_specs=[
            pl.BlockSpec((SB * B, N2V_DIM), lambda i: (i, 0)),
            col_spec(SMALL_D), col_spec(SMALL_D), col_spec(SMALL_D),
            col_spec(T_FEAT), col_spec(1),
            pl.BlockSpec((SB, 1, B), lambda i: (i, 0, 0)),
            full(wf), full(bf), full(popw), full(popb),
            full(w1t), full(w2t), full(w3t), full(w4t), full(w5t),
            full(w6t), full(projb),
        ],
        out_specs=pl.BlockSpec((SB, TARGET, B), lambda i: (i, 0, 0)),
        out_shape=jax.ShapeDtypeStruct((S, TARGET, B), jnp.float32),
        compiler_params=pltpu.CompilerParams(
            fuse_transposed_lhs_in_matmul=True),
    )(spat, btt, evt, eqt, tfT, popcT, countsT,
      wf, bf, popw, popb, w1t, w2t, w3t, w4t, w5t, w6t, projb)


def kernel(neighborhood_ids, time_features, building_type_ids,
           building_counts, population, event_type_ids, equipment_ids,
           node2vec_table, t2v_weight, t2v_bias, building_table,
           pop_W, pop_b, event_table, equip_table, proj_W, proj_b):
    idx_n2v = neighborhood_ids.reshape(B, S).T.reshape(N)
    idx_bt = building_type_ids[:, :, 0].T.reshape(N)
    idx_ev = event_type_ids.reshape(B, S).T.reshape(N)
    idx_eq = equipment_ids.reshape(B, S).T.reshape(N)

    spat, btt, evt, eqt = _sc_gather(
        node2vec_table,
        building_table.reshape(SMALL_V * SMALL_D),
        event_table.reshape(SMALL_V * SMALL_D),
        equip_table.reshape(SMALL_V * SMALL_D),
        idx_n2v, idx_bt, idx_ev, idx_eq)

    wt = proj_W.T
    out = _tc_fused(
        spat, btt, evt, eqt,
        jnp.transpose(time_features, (2, 1, 0)).reshape(T_FEAT, N),
        population[:, :, 0].T.reshape(1, N),
        building_counts[:, :, 0].T.reshape(S, 1, B),
        t2v_weight.reshape(T2V_DIM, 1),
        t2v_bias.reshape(T2V_DIM, 1),
        pop_W.reshape(8, 1), pop_b.reshape(8, 1),
        wt[:, 0:128], wt[:, 128:160], wt[:, 160:176],
        wt[:, 176:184], wt[:, 184:200], wt[:, 200:216],
        proj_b.reshape(TARGET, 1))
    return jnp.transpose(out, (2, 0, 1))
